# baseline (speedup 1.0000x reference)
import functools

import jax
import jax.numpy as jnp
import numpy as np
from jax.experimental import pallas as pl
from jax.experimental.pallas import tpu as pltpu

_G = 256
_SIGMA = 2.0
_TARGET = 1.0


def _build_smooth_matrix():
    k_size = int(6 * _SIGMA) | 1
    x = np.arange(k_size, dtype=np.float32) - k_size // 2
    k1 = np.exp(-(x ** 2) / (2.0 * _SIGMA ** 2))
    w = (k1 / k1.sum()).astype(np.float64)
    pad = k_size // 2
    s = np.zeros((_G, _G), dtype=np.float64)
    for t in range(k_size):
        off = t - pad
        for g in range(_G):
            i = g + off
            if i < 0:
                i = -i
            elif i >= _G:
                i = 2 * _G - 2 - i
            s[g, i] += w[t]
    return s.astype(np.float32)


_SMOOTH = _build_smooth_matrix()


def _split(x):
    hi = x.astype(jnp.bfloat16)
    lo = (x - hi.astype(jnp.float32)).astype(jnp.bfloat16)
    return hi, lo


_DIMS_NN = (((1,), (0,)), ((), ()))
_DIMS_NT = (((1,), (1,)), ((), ()))


def _dot2(a_hi, a_lo, b, dims):
    d = jax.lax.dot_general(a_lo, b, dims, preferred_element_type=jnp.float32)
    d += jax.lax.dot_general(a_hi, b, dims, preferred_element_type=jnp.float32)
    return d


def _body(px_ref, py_ref, ax_ref, ay_ref, sqh_ref, sql_ref, sh_ref, sl_ref,
          den_ref, loss_ref):
    g = _G
    nb = den_ref.shape[0]
    v = px_ref.shape[2]
    coords = jax.lax.broadcasted_iota(jnp.int32, (g, v), 0).astype(jnp.float32)
    for j in range(nb):
        gx = (px_ref[j] + 1.0) * ((g - 1) / 2.0)
        gy = (py_ref[j] + 1.0) * ((g - 1) / 2.0)
        x_in = 1.0 + jnp.tanh(ax_ref[0] - jnp.abs(coords - gx))
        y_in = 1.0 + jnp.tanh(ay_ref[0] - jnp.abs(coords - gy))
        yh, yl = _split(y_in)
        xb = x_in.astype(jnp.bfloat16)
        d = _dot2(yh, yl, xb, _DIMS_NT)
        t = _dot2(sqh_ref[...], sql_ref[...], d.astype(jnp.bfloat16),
                  _DIMS_NT)
        out = _dot2(sh_ref[...], sl_ref[...], t.astype(jnp.bfloat16),
                    _DIMS_NT)
        den_ref[j] = out
        ov = jnp.maximum(out - _TARGET, 0.0)
        part = jnp.sum(ov * ov, axis=0, keepdims=True)
        loss_ref[j] = part[:, :128] + part[:, 128:]


@jax.jit
def kernel(positions, sizes, macro_mask):
    b, v, _ = positions.shape
    g = _G
    px = positions[:, :, 0].reshape(b, 1, v)
    py = positions[:, :, 1].reshape(b, 1, v)
    mask = macro_mask
    ax = (sizes[:, 0] * (g / 4.0)).reshape(1, 1, v)
    ay = jnp.where(mask, sizes[:, 1] * (g / 4.0), -1e30).reshape(1, 1, v)
    smooth = jnp.asarray(_SMOOTH)
    smooth_q = smooth * 0.25
    smooth_q_hi = smooth_q.astype(jnp.bfloat16)
    smooth_q_lo = (smooth_q - smooth_q_hi.astype(jnp.float32)
                   ).astype(jnp.bfloat16)
    smooth_hi = smooth.astype(jnp.bfloat16)
    smooth_lo = (smooth - smooth_hi.astype(jnp.float32)).astype(jnp.bfloat16)

    nb = 2
    den, loss_part = pl.pallas_call(
        _body,
        grid=(b // nb,),
        in_specs=[
            pl.BlockSpec((nb, 1, v), lambda i: (i, 0, 0)),
            pl.BlockSpec((nb, 1, v), lambda i: (i, 0, 0)),
            pl.BlockSpec((1, 1, v), lambda i: (0, 0, 0)),
            pl.BlockSpec((1, 1, v), lambda i: (0, 0, 0)),
            pl.BlockSpec((g, g), lambda i: (0, 0)),
            pl.BlockSpec((g, g), lambda i: (0, 0)),
            pl.BlockSpec((g, g), lambda i: (0, 0)),
            pl.BlockSpec((g, g), lambda i: (0, 0)),
        ],
        out_specs=[
            pl.BlockSpec((nb, g, g), lambda i: (i, 0, 0)),
            pl.BlockSpec((nb, 1, 128), lambda i: (i, 0, 0)),
        ],
        out_shape=[
            jax.ShapeDtypeStruct((b, g, g), jnp.float32),
            jax.ShapeDtypeStruct((b, 1, 128), jnp.float32),
        ],
        compiler_params=pltpu.CompilerParams(
            dimension_semantics=("parallel",),
        ),
    )(px, py, ax, ay, smooth_q_hi, smooth_q_lo, smooth_hi, smooth_lo)

    density = den.reshape(b, 1, g, g)
    overflow_loss = jnp.sum(loss_part) / (b * g * g)
    return density, overflow_loss


# speedup vs baseline: 1.1144x; 1.1144x over previous
import functools

import jax
import jax.numpy as jnp
import numpy as np
from jax.experimental import pallas as pl
from jax.experimental.pallas import tpu as pltpu

_G = 256
_SIGMA = 2.0
_TARGET = 1.0


def _build_smooth_matrix():
    k_size = int(6 * _SIGMA) | 1
    x = np.arange(k_size, dtype=np.float32) - k_size // 2
    k1 = np.exp(-(x ** 2) / (2.0 * _SIGMA ** 2))
    w = (k1 / k1.sum()).astype(np.float64)
    pad = k_size // 2
    s = np.zeros((_G, _G), dtype=np.float64)
    for t in range(k_size):
        off = t - pad
        for g in range(_G):
            i = g + off
            if i < 0:
                i = -i
            elif i >= _G:
                i = 2 * _G - 2 - i
            s[g, i] += w[t]
    return s.astype(np.float32)


_SMOOTH = _build_smooth_matrix()


def _split(x):
    hi = x.astype(jnp.bfloat16)
    lo = (x - hi.astype(jnp.float32)).astype(jnp.bfloat16)
    return hi, lo


_DIMS_NN = (((1,), (0,)), ((), ()))
_DIMS_NT = (((1,), (1,)), ((), ()))


def _dot2(a_hi, a_lo, b, dims):
    d = jax.lax.dot_general(a_lo, b, dims, preferred_element_type=jnp.float32)
    d += jax.lax.dot_general(a_hi, b, dims, preferred_element_type=jnp.float32)
    return d


def _body(px_ref, py_ref, ax_ref, ay_ref, sqh_ref, sql_ref, sh_ref, sl_ref,
          den_ref, loss_ref):
    g = _G
    nb = den_ref.shape[0]
    v = px_ref.shape[2]
    coords = jax.lax.broadcasted_iota(jnp.int32, (g, v), 0).astype(jnp.float32)
    for j in range(nb):
        gx = (px_ref[j] + 1.0) * ((g - 1) / 2.0)
        gy = (py_ref[j] + 1.0) * ((g - 1) / 2.0)
        x_in = 1.0 + jnp.tanh(ax_ref[0] - jnp.abs(coords - gx))
        y_in = 1.0 + jnp.tanh(ay_ref[0] - jnp.abs(coords - gy))
        yh, yl = _split(y_in)
        xb = x_in.astype(jnp.bfloat16)
        d = _dot2(yh, yl, xb, _DIMS_NT)
        db = d.astype(jnp.bfloat16)
        t = _dot2(sqh_ref[...], sql_ref[...], db, _DIMS_NN)
        tb = t.astype(jnp.bfloat16)
        out = jax.lax.dot_general(tb, sl_ref[...], _DIMS_NT,
                                  preferred_element_type=jnp.float32)
        out += jax.lax.dot_general(tb, sh_ref[...], _DIMS_NT,
                                   preferred_element_type=jnp.float32)
        den_ref[j] = out
        ov = jnp.maximum(out - _TARGET, 0.0)
        part = jnp.sum(ov * ov, axis=0, keepdims=True)
        loss_ref[j] = part[:, :128] + part[:, 128:]


@jax.jit
def kernel(positions, sizes, macro_mask):
    b, v, _ = positions.shape
    g = _G
    px = positions[:, :, 0].reshape(b, 1, v)
    py = positions[:, :, 1].reshape(b, 1, v)
    mask = macro_mask
    ax = (sizes[:, 0] * (g / 4.0)).reshape(1, 1, v)
    ay = jnp.where(mask, sizes[:, 1] * (g / 4.0), -1e30).reshape(1, 1, v)
    smooth = jnp.asarray(_SMOOTH)
    smooth_q = smooth * 0.25
    smooth_q_hi = smooth_q.astype(jnp.bfloat16)
    smooth_q_lo = (smooth_q - smooth_q_hi.astype(jnp.float32)
                   ).astype(jnp.bfloat16)
    smooth_hi = smooth.astype(jnp.bfloat16)
    smooth_lo = (smooth - smooth_hi.astype(jnp.float32)).astype(jnp.bfloat16)

    nb = 1
    den, loss_part = pl.pallas_call(
        _body,
        grid=(b // nb,),
        in_specs=[
            pl.BlockSpec((nb, 1, v), lambda i: (i, 0, 0)),
            pl.BlockSpec((nb, 1, v), lambda i: (i, 0, 0)),
            pl.BlockSpec((1, 1, v), lambda i: (0, 0, 0)),
            pl.BlockSpec((1, 1, v), lambda i: (0, 0, 0)),
            pl.BlockSpec((g, g), lambda i: (0, 0)),
            pl.BlockSpec((g, g), lambda i: (0, 0)),
            pl.BlockSpec((g, g), lambda i: (0, 0)),
            pl.BlockSpec((g, g), lambda i: (0, 0)),
        ],
        out_specs=[
            pl.BlockSpec((nb, g, g), lambda i: (i, 0, 0)),
            pl.BlockSpec((nb, 1, 128), lambda i: (i, 0, 0)),
        ],
        out_shape=[
            jax.ShapeDtypeStruct((b, g, g), jnp.float32),
            jax.ShapeDtypeStruct((b, 1, 128), jnp.float32),
        ],
        compiler_params=pltpu.CompilerParams(
            dimension_semantics=("parallel",),
        ),
    )(px, py, ax, ay, smooth_q_hi, smooth_q_lo, smooth_hi, smooth_lo)

    density = den.reshape(b, 1, g, g)
    overflow_loss = jnp.sum(loss_part) / (b * g * g)
    return density, overflow_loss


# speedup vs baseline: 1.3469x; 1.2086x over previous
import functools

import jax
import jax.numpy as jnp
import numpy as np
from jax.experimental import pallas as pl
from jax.experimental.pallas import tpu as pltpu

_G = 256
_SIGMA = 2.0
_TARGET = 1.0


def _build_smooth_matrix():
    k_size = int(6 * _SIGMA) | 1
    x = np.arange(k_size, dtype=np.float32) - k_size // 2
    k1 = np.exp(-(x ** 2) / (2.0 * _SIGMA ** 2))
    w = (k1 / k1.sum()).astype(np.float64)
    pad = k_size // 2
    s = np.zeros((_G, _G), dtype=np.float64)
    for t in range(k_size):
        off = t - pad
        for g in range(_G):
            i = g + off
            if i < 0:
                i = -i
            elif i >= _G:
                i = 2 * _G - 2 - i
            s[g, i] += w[t]
    return s.astype(np.float32)


_SMOOTH = _build_smooth_matrix()


def _split(x):
    hi = x.astype(jnp.bfloat16)
    lo = (x - hi.astype(jnp.float32)).astype(jnp.bfloat16)
    return hi, lo


_DIMS_NN = (((1,), (0,)), ((), ()))
_DIMS_NT = (((1,), (1,)), ((), ()))


def _dot2(a_hi, a_lo, b, dims):
    d = jax.lax.dot_general(a_lo, b, dims, preferred_element_type=jnp.float32)
    d += jax.lax.dot_general(a_hi, b, dims, preferred_element_type=jnp.float32)
    return d


def _body(px_ref, py_ref, ax_ref, ay_ref, sqh_ref, sql_ref, sh_ref, sl_ref,
          den_ref, loss_ref, dscr_ref):
    g = _G
    v = px_ref.shape[2]
    i = pl.program_id(0)

    d_prev = dscr_ref[jax.lax.rem(i + 1, 2)]
    db = d_prev.astype(jnp.bfloat16)
    t = _dot2(sqh_ref[...], sql_ref[...], db, _DIMS_NN)
    tb = t.astype(jnp.bfloat16)
    out = jax.lax.dot_general(tb, sl_ref[...], _DIMS_NT,
                              preferred_element_type=jnp.float32)
    out += jax.lax.dot_general(tb, sh_ref[...], _DIMS_NT,
                               preferred_element_type=jnp.float32)
    den_ref[0] = out
    ov = jnp.maximum(out - _TARGET, 0.0)
    part = jnp.sum(ov * ov, axis=0, keepdims=True)
    loss_ref[0] = part[:, :128] + part[:, 128:]

    vc = 512
    coords = jax.lax.broadcasted_iota(jnp.int32, (g, vc), 0).astype(
        jnp.float32)
    d = None
    for c in range(v // vc):
        cs = slice(c * vc, (c + 1) * vc)
        gx = (px_ref[0][:, cs] + 1.0) * ((g - 1) / 2.0)
        gy = (py_ref[0][:, cs] + 1.0) * ((g - 1) / 2.0)
        x_in = 1.0 + jnp.tanh(ax_ref[0][:, cs] - jnp.abs(coords - gx))
        y_in = 1.0 + jnp.tanh(ay_ref[0][:, cs] - jnp.abs(coords - gy))
        yh, yl = _split(y_in)
        xb = x_in.astype(jnp.bfloat16)
        p = jax.lax.dot_general(yl, xb, _DIMS_NT,
                                preferred_element_type=jnp.float32)
        d = p if d is None else d + p
        d += jax.lax.dot_general(yh, xb, _DIMS_NT,
                                 preferred_element_type=jnp.float32)
    dscr_ref[jax.lax.rem(i, 2)] = d


@jax.jit
def kernel(positions, sizes, macro_mask):
    b, v, _ = positions.shape
    g = _G
    px = positions[:, :, 0].reshape(b, 1, v)
    py = positions[:, :, 1].reshape(b, 1, v)
    mask = macro_mask
    ax = (sizes[:, 0] * (g / 4.0)).reshape(1, 1, v)
    ay = jnp.where(mask, sizes[:, 1] * (g / 4.0), -1e30).reshape(1, 1, v)
    smooth = jnp.asarray(_SMOOTH)
    smooth_q = smooth * 0.25
    smooth_q_hi = smooth_q.astype(jnp.bfloat16)
    smooth_q_lo = (smooth_q - smooth_q_hi.astype(jnp.float32)
                   ).astype(jnp.bfloat16)
    smooth_hi = smooth.astype(jnp.bfloat16)
    smooth_lo = (smooth - smooth_hi.astype(jnp.float32)).astype(jnp.bfloat16)

    den, loss_part = pl.pallas_call(
        _body,
        grid=(b + 1,),
        in_specs=[
            pl.BlockSpec((1, 1, v), lambda i: (jnp.minimum(i, b - 1), 0, 0)),
            pl.BlockSpec((1, 1, v), lambda i: (jnp.minimum(i, b - 1), 0, 0)),
            pl.BlockSpec((1, 1, v), lambda i: (0, 0, 0)),
            pl.BlockSpec((1, 1, v), lambda i: (0, 0, 0)),
            pl.BlockSpec((g, g), lambda i: (0, 0)),
            pl.BlockSpec((g, g), lambda i: (0, 0)),
            pl.BlockSpec((g, g), lambda i: (0, 0)),
            pl.BlockSpec((g, g), lambda i: (0, 0)),
        ],
        out_specs=[
            pl.BlockSpec((1, g, g),
                         lambda i: (jnp.maximum(i - 1, 0), 0, 0)),
            pl.BlockSpec((1, 1, 128),
                         lambda i: (jnp.maximum(i - 1, 0), 0, 0)),
        ],
        out_shape=[
            jax.ShapeDtypeStruct((b, g, g), jnp.float32),
            jax.ShapeDtypeStruct((b, 1, 128), jnp.float32),
        ],
        scratch_shapes=[pltpu.VMEM((2, g, g), jnp.float32)],
        compiler_params=pltpu.CompilerParams(
            dimension_semantics=("arbitrary",),
        ),
    )(px, py, ax, ay, smooth_q_hi, smooth_q_lo, smooth_hi, smooth_lo)

    density = den.reshape(b, 1, g, g)
    overflow_loss = jnp.sum(loss_part) / (b * g * g)
    return density, overflow_loss


# speedup vs baseline: 1.4277x; 1.0600x over previous
import functools

import jax
import jax.numpy as jnp
import numpy as np
from jax.experimental import pallas as pl
from jax.experimental.pallas import tpu as pltpu

_G = 256
_SIGMA = 2.0
_TARGET = 1.0


def _build_smooth_matrix():
    k_size = int(6 * _SIGMA) | 1
    x = np.arange(k_size, dtype=np.float32) - k_size // 2
    k1 = np.exp(-(x ** 2) / (2.0 * _SIGMA ** 2))
    w = (k1 / k1.sum()).astype(np.float64)
    pad = k_size // 2
    s = np.zeros((_G, _G), dtype=np.float64)
    for t in range(k_size):
        off = t - pad
        for g in range(_G):
            i = g + off
            if i < 0:
                i = -i
            elif i >= _G:
                i = 2 * _G - 2 - i
            s[g, i] += w[t]
    return s.astype(np.float32)


_SMOOTH = _build_smooth_matrix()


def _split(x):
    hi = x.astype(jnp.bfloat16)
    lo = (x - hi.astype(jnp.float32)).astype(jnp.bfloat16)
    return hi, lo


_DIMS_NN = (((1,), (0,)), ((), ()))
_DIMS_NT = (((1,), (1,)), ((), ()))


def _dot2(a_hi, a_lo, b, dims):
    d = jax.lax.dot_general(a_lo, b, dims, preferred_element_type=jnp.float32)
    d += jax.lax.dot_general(a_hi, b, dims, preferred_element_type=jnp.float32)
    return d


def _body(px_ref, py_ref, ax_ref, ay_ref, sqh_ref, sql_ref, sh_ref, sl_ref,
          den_ref, loss_ref, dscr_ref):
    g = _G
    nb = den_ref.shape[0]
    v = px_ref.shape[2]
    i = pl.program_id(0)
    cur = jax.lax.rem(i, 2)
    prev = jax.lax.rem(i + 1, 2)

    for j in range(nb):
        db = dscr_ref[prev, j].astype(jnp.bfloat16)
        t = _dot2(sqh_ref[...], sql_ref[...], db, _DIMS_NN)
        tb = t.astype(jnp.bfloat16)
        out = jax.lax.dot_general(tb, sl_ref[...], _DIMS_NT,
                                  preferred_element_type=jnp.float32)
        out += jax.lax.dot_general(tb, sh_ref[...], _DIMS_NT,
                                   preferred_element_type=jnp.float32)
        den_ref[j] = out
        ov = jnp.maximum(out - _TARGET, 0.0)
        part = jnp.sum(ov * ov, axis=0, keepdims=True)
        loss_ref[j] = part[:, :128] + part[:, 128:]

    vc = 512
    coords = jax.lax.broadcasted_iota(jnp.int32, (g, vc), 0).astype(
        jnp.float32)
    for j in range(nb):
        d = None
        for c in range(v // vc):
            cs = slice(c * vc, (c + 1) * vc)
            gx = (px_ref[j][:, cs] + 1.0) * ((g - 1) / 2.0)
            gy = (py_ref[j][:, cs] + 1.0) * ((g - 1) / 2.0)
            x_in = 1.0 + jnp.tanh(ax_ref[0][:, cs] - jnp.abs(coords - gx))
            y_in = 1.0 + jnp.tanh(ay_ref[0][:, cs] - jnp.abs(coords - gy))
            yh, yl = _split(y_in)
            xb = x_in.astype(jnp.bfloat16)
            p = jax.lax.dot_general(yl, xb, _DIMS_NT,
                                    preferred_element_type=jnp.float32)
            d = p if d is None else d + p
            d += jax.lax.dot_general(yh, xb, _DIMS_NT,
                                     preferred_element_type=jnp.float32)
        dscr_ref[cur, j] = d


@jax.jit
def kernel(positions, sizes, macro_mask):
    b, v, _ = positions.shape
    g = _G
    px = positions[:, :, 0].reshape(b, 1, v)
    py = positions[:, :, 1].reshape(b, 1, v)
    mask = macro_mask
    ax = (sizes[:, 0] * (g / 4.0)).reshape(1, 1, v)
    ay = jnp.where(mask, sizes[:, 1] * (g / 4.0), -1e30).reshape(1, 1, v)
    smooth = jnp.asarray(_SMOOTH)
    smooth_q = smooth * 0.25
    smooth_q_hi = smooth_q.astype(jnp.bfloat16)
    smooth_q_lo = (smooth_q - smooth_q_hi.astype(jnp.float32)
                   ).astype(jnp.bfloat16)
    smooth_hi = smooth.astype(jnp.bfloat16)
    smooth_lo = (smooth - smooth_hi.astype(jnp.float32)).astype(jnp.bfloat16)

    nb = 2
    ns = b // nb
    den, loss_part = pl.pallas_call(
        _body,
        grid=(ns + 1,),
        in_specs=[
            pl.BlockSpec((nb, 1, v),
                         lambda i: (jnp.minimum(i, ns - 1), 0, 0)),
            pl.BlockSpec((nb, 1, v),
                         lambda i: (jnp.minimum(i, ns - 1), 0, 0)),
            pl.BlockSpec((1, 1, v), lambda i: (0, 0, 0)),
            pl.BlockSpec((1, 1, v), lambda i: (0, 0, 0)),
            pl.BlockSpec((g, g), lambda i: (0, 0)),
            pl.BlockSpec((g, g), lambda i: (0, 0)),
            pl.BlockSpec((g, g), lambda i: (0, 0)),
            pl.BlockSpec((g, g), lambda i: (0, 0)),
        ],
        out_specs=[
            pl.BlockSpec((nb, g, g),
                         lambda i: (jnp.maximum(i - 1, 0), 0, 0)),
            pl.BlockSpec((nb, 1, 128),
                         lambda i: (jnp.maximum(i - 1, 0), 0, 0)),
        ],
        out_shape=[
            jax.ShapeDtypeStruct((b, g, g), jnp.float32),
            jax.ShapeDtypeStruct((b, 1, 128), jnp.float32),
        ],
        scratch_shapes=[pltpu.VMEM((2, nb, g, g), jnp.float32)],
        compiler_params=pltpu.CompilerParams(
            dimension_semantics=("arbitrary",),
        ),
    )(px, py, ax, ay, smooth_q_hi, smooth_q_lo, smooth_hi, smooth_lo)

    density = den.reshape(b, 1, g, g)
    overflow_loss = jnp.sum(loss_part) / (b * g * g)
    return density, overflow_loss
